# Initial kernel scaffold; baseline (speedup 1.0000x reference)
import jax
import jax.numpy as jnp
from jax.experimental import pallas as pl


def kernel(x):
    raise NotImplementedError("write your pallas kernel here")


# speedup vs baseline: 2.1181x; 2.1181x over previous
import jax
import jax.numpy as jnp
from jax.experimental import pallas as pl


_ROW_BLOCK = 256
_CHUNK = 512


def _cumsum_kernel(x_ref, o_ref):
    rows, cols = x_ref.shape
    n_chunks = cols // _CHUNK
    row_i = jax.lax.broadcasted_iota(jnp.int32, (_CHUNK, _CHUNK), 0)
    col_i = jax.lax.broadcasted_iota(jnp.int32, (_CHUNK, _CHUNK), 1)
    tri = (row_i <= col_i).astype(jnp.float32)

    def body(k, carry):
        xk = x_ref[:, pl.ds(k * _CHUNK, _CHUNK)]
        sk = jax.lax.dot(xk, tri, precision=jax.lax.Precision.HIGHEST) + carry
        o_ref[:, pl.ds(k * _CHUNK, _CHUNK)] = sk
        return sk[:, _CHUNK - 1:_CHUNK]

    jax.lax.fori_loop(0, n_chunks, body, jnp.zeros((rows, 1), jnp.float32))


def kernel(x):
    n_rows, n_cols = x.shape
    grid = (n_rows // _ROW_BLOCK,)
    return pl.pallas_call(
        _cumsum_kernel,
        grid=grid,
        in_specs=[pl.BlockSpec((_ROW_BLOCK, n_cols), lambda i: (i, 0))],
        out_specs=pl.BlockSpec((_ROW_BLOCK, n_cols), lambda i: (i, 0)),
        out_shape=jax.ShapeDtypeStruct((n_rows, n_cols), x.dtype),
    )(x)
